# baseline (speedup 1.0000x reference)
import jax
import jax.numpy as jnp
from jax.experimental import pallas as pl
from jax.experimental.pallas import tpu as pltpu

D_MODEL = 1024
N_HEAD = 16
DH = 64
N_INTER = 4096
LORA_R = 8
LORA_SCALE = 2.0
N_EXPERTS = 4
B, S, L = 2, 2048, 256

_F32 = jnp.float32
_BF16 = jnp.bfloat16


def _router_kernel(x_ref, ln_ref, wr_ref, br_ref, a1_ref, comb_ref):
    x = x_ref[...]
    var = jnp.mean(x * x, axis=-1, keepdims=True)
    nx = x * jax.lax.rsqrt(var + 1e-6) * ln_ref[...][None]
    q_pool = jnp.mean(nx, axis=1)
    logits = jax.lax.dot_general(
        q_pool, wr_ref[...], (((1,), (0,)), ((), ())),
        preferred_element_type=_F32) + br_ref[...]
    aw = jax.nn.softmax(logits, axis=-1)
    idx = jax.lax.broadcasted_iota(jnp.int32, aw.shape, 1)
    big = jnp.int32(N_EXPERTS)
    w1 = jnp.max(aw, axis=-1, keepdims=True)
    i1 = jnp.min(jnp.where(aw >= w1, idx, big), axis=-1, keepdims=True)
    m = jnp.where(idx == i1, -jnp.inf, aw)
    w2 = jnp.max(m, axis=-1, keepdims=True)
    i2 = jnp.min(jnp.where(m >= w2, idx, big), axis=-1, keepdims=True)
    denom = w1 + w2 + 1e-10
    comb = jnp.where(idx == i1, w1, jnp.where(idx == i2, w2, 0.0)) / denom
    comb_ref[...] = comb * (1.0 / (1.0 + jnp.exp(-a1_ref[0, 0])))


def _router(x, ln1_w, wr, br, alpha_1):
    return pl.pallas_call(
        _router_kernel,
        in_specs=[
            pl.BlockSpec((B, S, D_MODEL), lambda: (0, 0, 0)),
            pl.BlockSpec((1, D_MODEL), lambda: (0, 0)),
            pl.BlockSpec((D_MODEL, N_EXPERTS), lambda: (0, 0)),
            pl.BlockSpec((1, N_EXPERTS), lambda: (0, 0)),
            pl.BlockSpec(memory_space=pltpu.SMEM),
        ],
        out_specs=pl.BlockSpec((B, N_EXPERTS), lambda: (0, 0)),
        out_shape=jax.ShapeDtypeStruct((B, N_EXPERTS), _F32),
    )(x, ln1_w.reshape(1, D_MODEL), wr, br.reshape(1, N_EXPERTS),
      alpha_1.reshape(1, 1))


_CS = 512


def _attn_kernel(comb_ref, x_ref, ln_ref, z_ref, wq_ref, wk_ref, wv_ref,
                 wo_ref, out_ref, k_s, v_s):
    b = pl.program_id(0)
    e = pl.program_id(1)

    @pl.when(e == 0)
    def _init():
        out_ref[...] = x_ref[...]

    w = comb_ref[b, e]

    @pl.when(w > 0.0)
    def _compute():
        z = z_ref[0, 0]
        k_s[...] = jnp.dot(z, wk_ref[0],
                           preferred_element_type=_F32).astype(_BF16)
        v_s[...] = jnp.dot(z, wv_ref[0],
                           preferred_element_type=_F32).astype(_BF16)
        ln = ln_ref[...]
        for c in range(S // _CS):
            rows = slice(c * _CS, (c + 1) * _CS)
            xc = x_ref[0, rows]
            var = jnp.mean(xc * xc, axis=-1, keepdims=True)
            nxc = (xc * jax.lax.rsqrt(var + 1e-6) * ln).astype(_BF16)
            qc = jnp.dot(nxc, wq_ref[0],
                         preferred_element_type=_F32).astype(_BF16)
            parts = []
            for h in range(N_HEAD):
                cols = slice(h * DH, (h + 1) * DH)
                s = jax.lax.dot_general(
                    qc[:, cols], k_s[:, cols], (((1,), (1,)), ((), ())),
                    preferred_element_type=_F32) * 0.125
                p = jax.nn.softmax(s, axis=-1).astype(_BF16)
                parts.append(jnp.dot(p, v_s[:, cols],
                                     preferred_element_type=_F32))
            o_cat = jnp.concatenate(parts, axis=1).astype(_BF16)
            out_ref[0, rows] += jnp.dot(
                o_cat, wo_ref[0], preferred_element_type=_F32) * w


def _attn(comb, x, ln1_w, zs, wqs, wks, wvs, wos):
    wspec = pl.BlockSpec((1, D_MODEL, D_MODEL), lambda b, e: (e, 0, 0))
    return pl.pallas_call(
        _attn_kernel,
        grid=(B, 3),
        in_specs=[
            pl.BlockSpec(memory_space=pltpu.SMEM),
            pl.BlockSpec((1, S, D_MODEL), lambda b, e: (b, 0, 0)),
            pl.BlockSpec((1, D_MODEL), lambda b, e: (0, 0)),
            pl.BlockSpec((1, 1, L, D_MODEL), lambda b, e: (e, b, 0, 0)),
            wspec, wspec, wspec, wspec,
        ],
        out_specs=pl.BlockSpec((1, S, D_MODEL), lambda b, e: (b, 0, 0)),
        out_shape=jax.ShapeDtypeStruct((B, S, D_MODEL), _F32),
        scratch_shapes=[
            pltpu.VMEM((L, D_MODEL), _BF16),
            pltpu.VMEM((L, D_MODEL), _BF16),
        ],
    )(comb, x, ln1_w.reshape(1, D_MODEL), zs, wqs, wks, wvs, wos)


_TB = 1024
_JB = 512
_NT = (B * S) // _TB
_NJ = N_INTER // _JB


def _mlp_kernel(x_ref, ln_ref, wg_ref, wu_ref, wd_ref,
                ag_ref, bg_ref, au_ref, bu_ref, ad_ref, bd_ref,
                a2_ref, out_ref,
                h_s, lg_s, lu_s, acc_s, tl_s):
    j = pl.program_id(1)

    @pl.when(j == 0)
    def _prep():
        x1 = x_ref[...]
        var = jnp.mean(x1 * x1, axis=-1, keepdims=True)
        h = x1 * jax.lax.rsqrt(var + 1e-6) * ln_ref[...]
        hb = h.astype(_BF16)
        h_s[...] = hb
        lg_s[...] = jnp.dot(hb, ag_ref[...],
                            preferred_element_type=_F32).astype(_BF16)
        lu_s[...] = jnp.dot(hb, au_ref[...],
                            preferred_element_type=_F32).astype(_BF16)
        acc_s[...] = jnp.zeros_like(acc_s)
        tl_s[...] = jnp.zeros_like(tl_s)

    hb = h_s[...]
    g = jnp.dot(hb, wg_ref[...], preferred_element_type=_F32)
    g += LORA_SCALE * jnp.dot(lg_s[...], bg_ref[...],
                              preferred_element_type=_F32)
    u = jnp.dot(hb, wu_ref[...], preferred_element_type=_F32)
    u += LORA_SCALE * jnp.dot(lu_s[...], bu_ref[...],
                              preferred_element_type=_F32)
    d = (g * jax.nn.sigmoid(g) + u).astype(_BF16)
    acc_s[...] += jnp.dot(d, wd_ref[...], preferred_element_type=_F32)
    tl_s[...] += jnp.dot(d, ad_ref[...], preferred_element_type=_F32)

    @pl.when(j == _NJ - 1)
    def _fin():
        mlp = acc_s[...] + LORA_SCALE * jnp.dot(
            tl_s[...].astype(_BF16), bd_ref[...], preferred_element_type=_F32)
        out_ref[...] = x_ref[...] + a2_ref[0, 0] * mlp


def _mlp(x2, ln2_w, wg, wu, wd, ag, bg, au, bu, ad, bd, a2):
    return pl.pallas_call(
        _mlp_kernel,
        grid=(_NT, _NJ),
        in_specs=[
            pl.BlockSpec((_TB, D_MODEL), lambda t, j: (t, 0)),
            pl.BlockSpec((1, D_MODEL), lambda t, j: (0, 0)),
            pl.BlockSpec((D_MODEL, _JB), lambda t, j: (0, j)),
            pl.BlockSpec((D_MODEL, _JB), lambda t, j: (0, j)),
            pl.BlockSpec((_JB, D_MODEL), lambda t, j: (j, 0)),
            pl.BlockSpec((D_MODEL, LORA_R), lambda t, j: (0, 0)),
            pl.BlockSpec((LORA_R, _JB), lambda t, j: (0, j)),
            pl.BlockSpec((D_MODEL, LORA_R), lambda t, j: (0, 0)),
            pl.BlockSpec((LORA_R, _JB), lambda t, j: (0, j)),
            pl.BlockSpec((_JB, LORA_R), lambda t, j: (j, 0)),
            pl.BlockSpec((LORA_R, D_MODEL), lambda t, j: (0, 0)),
            pl.BlockSpec(memory_space=pltpu.SMEM),
        ],
        out_specs=pl.BlockSpec((_TB, D_MODEL), lambda t, j: (t, 0)),
        out_shape=jax.ShapeDtypeStruct((B * S, D_MODEL), _F32),
        scratch_shapes=[
            pltpu.VMEM((_TB, D_MODEL), _BF16),
            pltpu.VMEM((_TB, LORA_R), _BF16),
            pltpu.VMEM((_TB, LORA_R), _BF16),
            pltpu.VMEM((_TB, D_MODEL), _F32),
            pltpu.VMEM((_TB, LORA_R), _F32),
        ],
    )(x2, ln2_w.reshape(1, D_MODEL), wg, wu, wd,
      ag, bg, au, bu, ad, bd, a2)


def kernel(x_q, z_a, z_v, z_av, params):
    p = params
    x = x_q[0]

    comb = _router(x, p['ln1_w'], p['Wr'], p['br'], p['alpha_1'])

    zs = jnp.stack([z_a, z_v, z_av]).astype(_BF16)
    wqs = jnp.stack([p['Wq_a'], p['Wq_v'], p['Wq_av']]).astype(_BF16)
    wks = jnp.stack([p['Wk_a'], p['Wk_v'], p['Wk_av']]).astype(_BF16)
    wvs = jnp.stack([p['Wv_a'], p['Wv_v'], p['Wv_av']]).astype(_BF16)
    wos = jnp.stack([p['Wo_a'], p['Wo_v'], p['Wo_av']]).astype(_BF16)
    x1 = _attn(comb, x, p['ln1_w'], zs, wqs, wks, wvs, wos)

    a2 = jax.nn.sigmoid(p['alpha_2']).reshape(1, 1)
    out = _mlp(
        x1.reshape(B * S, D_MODEL),
        p['ln2_w'],
        p['Wg'].astype(_BF16), p['Wu'].astype(_BF16), p['Wd'].astype(_BF16),
        p['Ag'].astype(_BF16), p['Bg'].astype(_BF16),
        p['Au'].astype(_BF16), p['Bu'].astype(_BF16),
        p['Ad'].astype(_BF16), p['Bd'].astype(_BF16),
        a2)
    return out.reshape(B, S, D_MODEL)
